# baseline (speedup 1.0000x reference)
import functools

import jax
import jax.numpy as jnp
from jax import lax
from jax.experimental import pallas as pl
from jax.experimental.pallas import tpu as pltpu
from jax.experimental.pallas import tpu_sc as plsc

D = 32
B_TOTAL = 16384 * 200

NC, NS = 2, 16
NW = NC * NS
B_PER_W = B_TOTAL // NW
CHUNK = 1024
N_CHUNKS = B_PER_W // CHUNK
NBUF = 3

_mesh = plsc.VectorSubcoreMesh(core_axis_name="c", subcore_axis_name="s")

VOCAB = 1_000_000
N_COLS = VOCAB // 128
TAIL = VOCAB - N_COLS * 128
TAIL_OWNER = N_COLS % NW


@functools.partial(
    pl.kernel,
    mesh=_mesh,
    out_type=jax.ShapeDtypeStruct((VOCAB // 4, 128), jnp.float32),
    scratch_types=[
        [pltpu.VMEM((D, 128), jnp.float32) for _ in range(2)],
        [pltpu.VMEM((32, 128), jnp.float32) for _ in range(2)],
        [pltpu.SemaphoreType.DMA for _ in range(2)],
        [pltpu.SemaphoreType.DMA for _ in range(2)],
    ],
    compiler_params=pltpu.CompilerParams(use_tc_tiling_on_sc=True, needs_layout_passes=False),
)
def _transpose_kernel(tbl_hbm, tail_hbm, out_hbm, stage_in, stage_out, sin, sout):
    wid = lax.axis_index("s") * NC + lax.axis_index("c")
    lane = lax.iota(jnp.int32, 16)
    d_lo = lane
    d_hi = lane + 16

    def col_of(t, b):
        return (2 * t + b) * NW + wid

    def start_in(j, b):
        pltpu.async_copy(
            tbl_hbm.at[:, pl.ds(j * 128, 128)], stage_in[b], sin[b])

    def wait_in(j, b):
        pltpu.make_async_copy(
            tbl_hbm.at[:, pl.ds(j * 128, 128)], stage_in[b], sin[b]).wait()

    def start_out(j, b):
        pltpu.async_copy(
            stage_out[b], out_hbm.at[pl.ds(j * 32, 32), :], sout[b])

    def wait_out(j, b):
        pltpu.make_async_copy(
            stage_out[b], out_hbm.at[pl.ds(j * 32, 32), :], sout[b]).wait()

    def permute(b):
        for sr in range(32):
            for k in range(8):
                d_vec = d_lo if (k % 2) == 0 else d_hi
                v_vec = jnp.full((16,), 4 * sr + k // 2, jnp.int32)
                vals = plsc.load_gather(stage_in[b], [d_vec, v_vec])
                stage_out[b][sr, pl.ds(16 * k, 16)] = vals

    n_iter = (N_COLS - wid + NW - 1) // NW

    @pl.when(n_iter >= 1)
    def _():
        start_in(col_of(0, 0), 0)

    @pl.when(n_iter >= 2)
    def _():
        start_in(col_of(0, 1), 1)

    def body(t, carry):
        for b in range(2):
            j = col_of(t, b)

            @pl.when(j < N_COLS)
            def _():
                wait_in(j, b)

                @pl.when(t >= 1)
                def _():
                    wait_out(j - 2 * NW, b)

                permute(b)
                start_out(j, b)
                jn = j + 2 * NW

                @pl.when(jn < N_COLS)
                def _():
                    start_in(jn, b)
        return carry

    lax.fori_loop(0, (n_iter + 1) // 2, body, 0, unroll=False)

    for b in range(2):
        @pl.when(jnp.logical_and(n_iter >= 1, (n_iter - 1) % 2 == b))
        def _(b=b):
            wait_out((n_iter - 1) * NW + wid, b)

        @pl.when(jnp.logical_and(n_iter >= 2, (n_iter - 2) % 2 == b))
        def _(b=b):
            wait_out((n_iter - 2) * NW + wid, b)

    @pl.when(wid == TAIL_OWNER)
    def _():
        pltpu.sync_copy(tail_hbm, stage_out[0].at[pl.ds(0, TAIL // 4), :])
        pltpu.sync_copy(
            stage_out[0].at[pl.ds(0, TAIL // 4), :],
            out_hbm.at[pl.ds(N_COLS * 32, TAIL // 4), :])


@functools.partial(
    pl.kernel,
    mesh=_mesh,
    out_type=jax.ShapeDtypeStruct((B_TOTAL, D), jnp.float32),
    scratch_types=[
        [pltpu.VMEM((CHUNK,), jnp.int32) for _ in range(NBUF)],
        [pltpu.VMEM((CHUNK, D), jnp.float32) for _ in range(NBUF)],
        [pltpu.SemaphoreType.DMA for _ in range(NBUF)],
        [pltpu.SemaphoreType.DMA for _ in range(NBUF)],
        [pltpu.SemaphoreType.DMA for _ in range(NBUF)],
    ],
    compiler_params=pltpu.CompilerParams(use_tc_tiling_on_sc=False),
)
def _gather_kernel(idx_hbm, table_hbm, out_hbm, idx_v, rows_v, si, sg, so):
    wid = lax.axis_index("s") * NC + lax.axis_index("c")
    base = wid * B_PER_W

    def start_idx(i, b):
        pltpu.async_copy(
            idx_hbm.at[pl.ds(base + i * CHUNK, CHUNK)], idx_v[b], si[b])

    def wait_idx(i, b):
        pltpu.make_async_copy(
            idx_hbm.at[pl.ds(base + i * CHUNK, CHUNK)], idx_v[b], si[b]).wait()

    def start_gather(b):
        pltpu.async_copy(table_hbm.at[idx_v[b]], rows_v[b], sg[b])

    def wait_gather(b):
        pltpu.make_async_copy(table_hbm.at[idx_v[b]], rows_v[b], sg[b]).wait()

    def start_out(i, b):
        pltpu.async_copy(
            rows_v[b], out_hbm.at[pl.ds(base + i * CHUNK, CHUNK)], so[b])

    def wait_out(i, b):
        pltpu.make_async_copy(
            rows_v[b], out_hbm.at[pl.ds(base + i * CHUNK, CHUNK)], so[b]).wait()

    start_idx(0, 0)
    start_idx(1, 1)
    wait_idx(0, 0)
    start_gather(0)

    def outer(g, carry):
        for bb in range(NBUF):
            i = g * NBUF + bb
            b = bb
            nb = (bb + 1) % NBUF
            pb = (bb + 2) % NBUF

            @pl.when(i + 1 < N_CHUNKS)
            def _():
                wait_idx(i + 1, nb)
                @pl.when(i + 1 >= NBUF)
                def _():
                    wait_out(i + 1 - NBUF, nb)
                start_gather(nb)

            @pl.when(i + 2 < N_CHUNKS)
            def _():
                start_idx(i + 2, pb)

            wait_gather(b)
            start_out(i, b)
        return carry

    lax.fori_loop(0, N_CHUNKS // NBUF, outer, 0, unroll=False)

    for i in range(N_CHUNKS - N_CHUNKS % NBUF, N_CHUNKS):
        b = i % NBUF
        nb = (i + 1) % NBUF
        if i + 1 < N_CHUNKS:
            wait_idx(i + 1, nb)
            wait_out(i + 1 - NBUF, nb)
            start_gather(nb)
        if i + 2 < N_CHUNKS:
            start_idx(i + 2, (i + 2) % NBUF)
        wait_gather(b)
        start_out(i, b)

    for i in range(max(0, N_CHUNKS - NBUF), N_CHUNKS):
        wait_out(i, i % NBUF)


def kernel(x, table):
    idx = x.reshape(-1).astype(jnp.int32)
    tail_lin = table[N_COLS * 128:, :].reshape(TAIL // 4, 128)
    tbl_lin = _transpose_kernel(table.T, tail_lin).reshape(VOCAB, D)
    out = _gather_kernel(idx, tbl_lin)
    return out.reshape(x.shape[0], x.shape[1], D)


# speedup vs baseline: 1.0914x; 1.0914x over previous
import functools

import jax
import jax.numpy as jnp
from jax import lax
from jax.experimental import pallas as pl
from jax.experimental.pallas import tpu as pltpu
from jax.experimental.pallas import tpu_sc as plsc

D = 32
B_TOTAL = 16384 * 200

NC, NS = 2, 16
NW = NC * NS
B_PER_W = B_TOTAL // NW
CHUNK = 1024
N_CHUNKS = B_PER_W // CHUNK
NBUF = 3

_mesh = plsc.VectorSubcoreMesh(core_axis_name="c", subcore_axis_name="s")

VOCAB = 1_000_000
N_COLS = VOCAB // 128
TAIL = VOCAB - N_COLS * 128
TAIL_OWNER = N_COLS % NW


@functools.partial(
    pl.kernel,
    mesh=_mesh,
    out_type=jax.ShapeDtypeStruct((VOCAB // 4, 128), jnp.float32),
    scratch_types=[
        [pltpu.VMEM((D, 128), jnp.float32) for _ in range(2)],
        [pltpu.VMEM((32, 128), jnp.float32) for _ in range(2)],
        [pltpu.SemaphoreType.DMA for _ in range(2)],
        [pltpu.SemaphoreType.DMA for _ in range(2)],
    ],
    compiler_params=pltpu.CompilerParams(use_tc_tiling_on_sc=True, needs_layout_passes=False),
)
def _transpose_kernel(tbl_hbm, tail_hbm, out_hbm, stage_in, stage_out, sin, sout):
    wid = lax.axis_index("s") * NC + lax.axis_index("c")
    lane = lax.iota(jnp.int32, 16)
    d_lo = lane
    d_hi = lane + 16

    def col_of(t, b):
        return (2 * t + b) * NW + wid

    def start_in(j, b):
        pltpu.async_copy(
            tbl_hbm.at[:, pl.ds(j * 128, 128)], stage_in[b], sin[b])

    def wait_in(j, b):
        pltpu.make_async_copy(
            tbl_hbm.at[:, pl.ds(j * 128, 128)], stage_in[b], sin[b]).wait()

    def start_out(j, b):
        pltpu.async_copy(
            stage_out[b], out_hbm.at[pl.ds(j * 32, 32), :], sout[b])

    def wait_out(j, b):
        pltpu.make_async_copy(
            stage_out[b], out_hbm.at[pl.ds(j * 32, 32), :], sout[b]).wait()

    def permute(b):
        for sr in range(32):
            vals = []
            for k in range(8):
                d_vec = d_lo if (k % 2) == 0 else d_hi
                v_vec = jnp.full((16,), 4 * sr + k // 2, jnp.int32)
                vals.append(plsc.load_gather(stage_in[b], [d_vec, v_vec]))
            for k in range(8):
                stage_out[b][sr, pl.ds(16 * k, 16)] = vals[k]

    n_iter = (N_COLS - wid + NW - 1) // NW

    @pl.when(n_iter >= 1)
    def _():
        start_in(col_of(0, 0), 0)

    @pl.when(n_iter >= 2)
    def _():
        start_in(col_of(0, 1), 1)

    def body(t, carry):
        for b in range(2):
            j = col_of(t, b)

            @pl.when(j < N_COLS)
            def _():
                wait_in(j, b)

                @pl.when(t >= 1)
                def _():
                    wait_out(j - 2 * NW, b)

                permute(b)
                start_out(j, b)
                jn = j + 2 * NW

                @pl.when(jn < N_COLS)
                def _():
                    start_in(jn, b)
        return carry

    lax.fori_loop(0, (n_iter + 1) // 2, body, 0, unroll=False)

    for b in range(2):
        @pl.when(jnp.logical_and(n_iter >= 1, (n_iter - 1) % 2 == b))
        def _(b=b):
            wait_out((n_iter - 1) * NW + wid, b)

        @pl.when(jnp.logical_and(n_iter >= 2, (n_iter - 2) % 2 == b))
        def _(b=b):
            wait_out((n_iter - 2) * NW + wid, b)

    @pl.when(wid == TAIL_OWNER)
    def _():
        pltpu.sync_copy(tail_hbm, stage_out[0].at[pl.ds(0, TAIL // 4), :])
        pltpu.sync_copy(
            stage_out[0].at[pl.ds(0, TAIL // 4), :],
            out_hbm.at[pl.ds(N_COLS * 32, TAIL // 4), :])


@functools.partial(
    pl.kernel,
    mesh=_mesh,
    out_type=jax.ShapeDtypeStruct((B_TOTAL, D), jnp.float32),
    scratch_types=[
        [pltpu.VMEM((CHUNK,), jnp.int32) for _ in range(NBUF)],
        [pltpu.VMEM((CHUNK, D), jnp.float32) for _ in range(NBUF)],
        [pltpu.SemaphoreType.DMA for _ in range(NBUF)],
        [pltpu.SemaphoreType.DMA for _ in range(NBUF)],
        [pltpu.SemaphoreType.DMA for _ in range(NBUF)],
    ],
    compiler_params=pltpu.CompilerParams(use_tc_tiling_on_sc=False),
)
def _gather_kernel(idx_hbm, table_hbm, out_hbm, idx_v, rows_v, si, sg, so):
    wid = lax.axis_index("s") * NC + lax.axis_index("c")
    base = wid * B_PER_W

    def start_idx(i, b):
        pltpu.async_copy(
            idx_hbm.at[pl.ds(base + i * CHUNK, CHUNK)], idx_v[b], si[b])

    def wait_idx(i, b):
        pltpu.make_async_copy(
            idx_hbm.at[pl.ds(base + i * CHUNK, CHUNK)], idx_v[b], si[b]).wait()

    def start_gather(b):
        pltpu.async_copy(table_hbm.at[idx_v[b]], rows_v[b], sg[b])

    def wait_gather(b):
        pltpu.make_async_copy(table_hbm.at[idx_v[b]], rows_v[b], sg[b]).wait()

    def start_out(i, b):
        pltpu.async_copy(
            rows_v[b], out_hbm.at[pl.ds(base + i * CHUNK, CHUNK)], so[b])

    def wait_out(i, b):
        pltpu.make_async_copy(
            rows_v[b], out_hbm.at[pl.ds(base + i * CHUNK, CHUNK)], so[b]).wait()

    start_idx(0, 0)
    start_idx(1, 1)
    wait_idx(0, 0)
    start_gather(0)

    def outer(g, carry):
        for bb in range(NBUF):
            i = g * NBUF + bb
            b = bb
            nb = (bb + 1) % NBUF
            pb = (bb + 2) % NBUF

            @pl.when(i + 1 < N_CHUNKS)
            def _():
                wait_idx(i + 1, nb)
                @pl.when(i + 1 >= NBUF)
                def _():
                    wait_out(i + 1 - NBUF, nb)
                start_gather(nb)

            @pl.when(i + 2 < N_CHUNKS)
            def _():
                start_idx(i + 2, pb)

            wait_gather(b)
            start_out(i, b)
        return carry

    lax.fori_loop(0, N_CHUNKS // NBUF, outer, 0, unroll=False)

    for i in range(N_CHUNKS - N_CHUNKS % NBUF, N_CHUNKS):
        b = i % NBUF
        nb = (i + 1) % NBUF
        if i + 1 < N_CHUNKS:
            wait_idx(i + 1, nb)
            wait_out(i + 1 - NBUF, nb)
            start_gather(nb)
        if i + 2 < N_CHUNKS:
            start_idx(i + 2, (i + 2) % NBUF)
        wait_gather(b)
        start_out(i, b)

    for i in range(max(0, N_CHUNKS - NBUF), N_CHUNKS):
        wait_out(i, i % NBUF)


def kernel(x, table):
    idx = x.reshape(-1).astype(jnp.int32)
    tail_lin = table[N_COLS * 128:, :].reshape(TAIL // 4, 128)
    tbl_lin = _transpose_kernel(table.T, tail_lin).reshape(VOCAB, D)
    out = _gather_kernel(idx, tbl_lin)
    return out.reshape(x.shape[0], x.shape[1], D)


# speedup vs baseline: 1.1115x; 1.0184x over previous
import functools

import jax
import jax.numpy as jnp
from jax import lax
from jax.experimental import pallas as pl
from jax.experimental.pallas import tpu as pltpu
from jax.experimental.pallas import tpu_sc as plsc

D = 32
B_TOTAL = 16384 * 200

NC, NS = 2, 16
NW = NC * NS
B_PER_W = B_TOTAL // NW
CHUNK = 1024
N_CHUNKS = B_PER_W // CHUNK
NBUF = 3

_mesh = plsc.VectorSubcoreMesh(core_axis_name="c", subcore_axis_name="s")


@functools.partial(
    pl.kernel,
    mesh=_mesh,
    out_type=jax.ShapeDtypeStruct((B_TOTAL, D), jnp.float32),
    scratch_types=[
        [pltpu.VMEM((CHUNK,), jnp.int32) for _ in range(NBUF)],
        [pltpu.VMEM((CHUNK, D), jnp.float32) for _ in range(NBUF)],
        [pltpu.SemaphoreType.DMA for _ in range(NBUF)],
        [pltpu.SemaphoreType.DMA for _ in range(NBUF)],
        [pltpu.SemaphoreType.DMA for _ in range(NBUF)],
    ],
    compiler_params=pltpu.CompilerParams(use_tc_tiling_on_sc=False),
)
def _gather_kernel(idx_hbm, table_hbm, out_hbm, idx_v, rows_v, si, sg, so):
    wid = lax.axis_index("s") * NC + lax.axis_index("c")
    base = wid * B_PER_W

    def start_idx(i, b):
        pltpu.async_copy(
            idx_hbm.at[pl.ds(base + i * CHUNK, CHUNK)], idx_v[b], si[b])

    def wait_idx(i, b):
        pltpu.make_async_copy(
            idx_hbm.at[pl.ds(base + i * CHUNK, CHUNK)], idx_v[b], si[b]).wait()

    def start_gather(b):
        pltpu.async_copy(table_hbm.at[idx_v[b]], rows_v[b], sg[b])

    def wait_gather(b):
        pltpu.make_async_copy(table_hbm.at[idx_v[b]], rows_v[b], sg[b]).wait()

    def start_out(i, b):
        pltpu.async_copy(
            rows_v[b], out_hbm.at[pl.ds(base + i * CHUNK, CHUNK)], so[b])

    def wait_out(i, b):
        pltpu.make_async_copy(
            rows_v[b], out_hbm.at[pl.ds(base + i * CHUNK, CHUNK)], so[b]).wait()

    start_idx(0, 0)
    start_idx(1, 1)
    wait_idx(0, 0)
    start_gather(0)

    def outer(g, carry):
        for bb in range(NBUF):
            i = g * NBUF + bb
            b = bb
            nb = (bb + 1) % NBUF
            pb = (bb + 2) % NBUF

            @pl.when(i + 1 < N_CHUNKS)
            def _():
                wait_idx(i + 1, nb)
                @pl.when(i + 1 >= NBUF)
                def _():
                    wait_out(i + 1 - NBUF, nb)
                start_gather(nb)

            @pl.when(i + 2 < N_CHUNKS)
            def _():
                start_idx(i + 2, pb)

            wait_gather(b)
            start_out(i, b)
        return carry

    lax.fori_loop(0, N_CHUNKS // NBUF, outer, 0, unroll=False)

    for i in range(N_CHUNKS - N_CHUNKS % NBUF, N_CHUNKS):
        b = i % NBUF
        nb = (i + 1) % NBUF
        if i + 1 < N_CHUNKS:
            wait_idx(i + 1, nb)
            wait_out(i + 1 - NBUF, nb)
            start_gather(nb)
        if i + 2 < N_CHUNKS:
            start_idx(i + 2, (i + 2) % NBUF)
        wait_gather(b)
        start_out(i, b)

    for i in range(max(0, N_CHUNKS - NBUF), N_CHUNKS):
        wait_out(i, i % NBUF)


def kernel(x, table):
    idx = x.reshape(-1).astype(jnp.int32)
    out = _gather_kernel(idx, table)
    return out.reshape(x.shape[0], x.shape[1], D)
